# baseline (speedup 1.0000x reference)
import jax
import jax.numpy as jnp
from jax import lax
from jax.experimental import pallas as pl
from jax.experimental.pallas import tpu as pltpu

R = 10
_CH = 512
_NBUF = 2


def _stream_kernel(len_ref, w_ref, x_ref, o_ref, xbuf, sem0, sem1):
    sems = (sem0, sem1)
    NROWS = x_ref.shape[0]
    NCH = NROWS // _CH
    cpb = 4096 // _CH
    w = w_ref[...]

    def copy(g):
        return pltpu.make_async_copy(
            x_ref.at[pl.ds(g * _CH, _CH), :], xbuf.at[g % _NBUF],
            sems[g % _NBUF])

    copy(0).start()
    smax = [None] * R
    smin = [None] * R
    for g in range(NCH):
        b, c = divmod(g, cpb)
        if g + 1 < NCH:
            copy(g + 1).start()
        copy(g).wait()

        s = jnp.dot(xbuf[g % _NBUF], w,
                    preferred_element_type=jnp.float32).T
        t = c * _CH + lax.broadcasted_iota(jnp.int32, (1, _CH), 1)
        L = len_ref[b]

        for r in range(R):
            start = (r * L) // R
            end = ((r + 1) * L + R - 1) // R
            mask = (t >= start) & (t < end)
            cmax = jnp.max(jnp.where(mask, s, -jnp.inf))
            cmin = jnp.min(jnp.where(mask, s, jnp.inf))
            if c == 0:
                smax[r] = cmax
                smin[r] = cmin
            else:
                smax[r] = jnp.maximum(smax[r], cmax)
                smin[r] = jnp.minimum(smin[r], cmin)

        if c == cpb - 1:
            acc = jnp.float32(0.0)
            for r in range(R):
                acc = acc + smax[r] + smin[r]
            o_ref[b, 0, :] = jnp.full((128,), jax.nn.sigmoid(acc),
                                      dtype=jnp.float32)


@jax.jit
def kernel(x, lengths, W):
    B, T, D = x.shape
    xf = x.reshape(B * T, D)
    pooled = pl.pallas_call(
        _stream_kernel,
        in_specs=[
            pl.BlockSpec(memory_space=pltpu.SMEM),
            pl.BlockSpec(memory_space=pltpu.VMEM),
            pl.BlockSpec(memory_space=pl.ANY),
        ],
        out_specs=pl.BlockSpec(memory_space=pltpu.VMEM),
        out_shape=jax.ShapeDtypeStruct((B, 1, 128), jnp.float32),
        scratch_shapes=[
            pltpu.VMEM((_NBUF, _CH, D), jnp.float32),
            pltpu.SemaphoreType.DMA,
            pltpu.SemaphoreType.DMA,
        ],
    )(lengths, W, xf)
    return pooled[:, 0, 0]


# speedup vs baseline: 1.4145x; 1.4145x over previous
import jax
import jax.numpy as jnp
from jax.experimental import pallas as pl
from jax.experimental.pallas import tpu as pltpu

R = 10
_TB = 2048


def _fused_kernel(len_ref, x_ref, w_ref, o_ref, smax_ref, smin_ref):
    i = pl.program_id(0)
    nchunks = pl.num_programs(0)
    TB = x_ref.shape[0]

    s = jnp.dot(x_ref[...], w_ref[...],
                preferred_element_type=jnp.float32).T

    cpb = nchunks // len_ref.shape[0]
    b = i // cpb
    c = i % cpb
    off = c * TB
    L = len_ref[b]

    t = off + jax.lax.broadcasted_iota(jnp.int32, (1, TB), 1)

    @pl.when(c == 0)
    def _init():
        for r in range(R):
            smax_ref[r] = jnp.float32(-jnp.inf)
            smin_ref[r] = jnp.float32(jnp.inf)

    for r in range(R):
        start = (r * L) // R
        end = ((r + 1) * L + R - 1) // R
        mask = (t >= start) & (t < end)
        cmax = jnp.max(jnp.where(mask, s, -jnp.inf))
        cmin = jnp.min(jnp.where(mask, s, jnp.inf))
        smax_ref[r] = jnp.maximum(smax_ref[r], cmax)
        smin_ref[r] = jnp.minimum(smin_ref[r], cmin)

    @pl.when(c == cpb - 1)
    def _finish():
        acc = jnp.float32(0.0)
        for r in range(R):
            acc = acc + smax_ref[r] + smin_ref[r]
        o_ref[0, 0, :] = jnp.full((128,), jax.nn.sigmoid(acc),
                                  dtype=jnp.float32)


@jax.jit
def kernel(x, lengths, W):
    B, T, D = x.shape
    nt = T // _TB
    xf = x.reshape(B * T, D)
    pooled = pl.pallas_call(
        _fused_kernel,
        grid=(B * T // _TB,),
        in_specs=[
            pl.BlockSpec(memory_space=pltpu.SMEM),
            pl.BlockSpec((_TB, D), lambda i: (i, 0)),
            pl.BlockSpec((D, 1), lambda i: (0, 0)),
        ],
        out_specs=pl.BlockSpec((1, 1, 128), lambda i: (i // nt, 0, 0)),
        out_shape=jax.ShapeDtypeStruct((B, 1, 128), jnp.float32),
        scratch_shapes=[
            pltpu.SMEM((R,), jnp.float32),
            pltpu.SMEM((R,), jnp.float32),
        ],
    )(lengths, xf, W)
    return pooled[:, 0, 0]


# speedup vs baseline: 1.4156x; 1.0008x over previous
import jax
import jax.numpy as jnp
from jax.experimental import pallas as pl
from jax.experimental.pallas import tpu as pltpu

R = 10
_TB = 2048


def _fused_kernel(len_ref, x_ref, w_ref, o_ref, smax_ref, smin_ref):
    i = pl.program_id(0)
    nchunks = pl.num_programs(0)
    TB = x_ref.shape[0]

    s = jnp.dot(x_ref[...], w_ref[...],
                preferred_element_type=jnp.float32).T

    cpb = nchunks // len_ref.shape[0]
    b = i // cpb
    c = i % cpb
    off = c * TB
    L = len_ref[b]

    t = off + jax.lax.broadcasted_iota(jnp.int32, (1, TB), 1)

    @pl.when(c == 0)
    def _init():
        for r in range(R):
            smax_ref[r] = jnp.float32(-jnp.inf)
            smin_ref[r] = jnp.float32(jnp.inf)

    for r in range(R):
        start = (r * L) // R
        end = ((r + 1) * L + R - 1) // R
        mask = (t >= start) & (t < end)
        cmax = jnp.max(jnp.where(mask, s, -jnp.inf))
        cmin = jnp.min(jnp.where(mask, s, jnp.inf))
        smax_ref[r] = jnp.maximum(smax_ref[r], cmax)
        smin_ref[r] = jnp.minimum(smin_ref[r], cmin)

    @pl.when(c == cpb - 1)
    def _finish():
        acc = jnp.float32(0.0)
        for r in range(R):
            acc = acc + smax_ref[r] + smin_ref[r]
        o_ref[0, 0, :] = jnp.full((128,), jax.nn.sigmoid(acc),
                                  dtype=jnp.float32)


@jax.jit
def kernel(x, lengths, W):
    B, T, D = x.shape
    nt = T // _TB
    xf = x.reshape(B * T, D)
    pooled = pl.pallas_call(
        _fused_kernel,
        grid=(B * T // _TB,),
        in_specs=[
            pl.BlockSpec(memory_space=pltpu.SMEM),
            pl.BlockSpec((_TB, D), lambda i: (i, 0)),
            pl.BlockSpec((D, 1), lambda i: (0, 0)),
        ],
        out_specs=pl.BlockSpec((1, 1, 128), lambda i: (i // nt, 0, 0)),
        out_shape=jax.ShapeDtypeStruct((B, 1, 128), jnp.float32),
        scratch_shapes=[
            pltpu.SMEM((R,), jnp.float32),
            pltpu.SMEM((R,), jnp.float32),
        ],
        compiler_params=pltpu.CompilerParams(
            dimension_semantics=("arbitrary",)),
    )(lengths, xf, W)
    return pooled[:, 0, 0]
